# Initial kernel scaffold; baseline (speedup 1.0000x reference)
import jax
import jax.numpy as jnp
from jax.experimental import pallas as pl


def kernel(x, tables, W1, b1, W2, b2):
    raise NotImplementedError("write your pallas kernel here")


# speedup vs baseline: 1.0379x; 1.0379x over previous
import functools

import numpy as np
import jax
import jax.numpy as jnp
from jax import lax
from jax.experimental import pallas as pl
from jax.experimental.pallas import tpu as pltpu
from jax.experimental.pallas import tpu_sc as plsc

B = 16384
N_DENSE = 13
N_SPARSE = 26
K = 16
VOCAB = 100000
SPARSE_DIMS = N_SPARSE * K
N_PAIRS = N_SPARSE * (N_SPARSE - 1) // 2
DNN_IN = N_PAIRS + SPARSE_DIMS + N_DENSE

_row, _col = np.triu_indices(N_SPARSE, k=1)
_TRIU_POS = {(int(r), int(c)): t for t, (r, c) in enumerate(zip(_row, _col))}
_BAND_PAIRS = [(i, i + d) for d in range(1, N_SPARSE) for i in range(N_SPARSE - d)]
_BAND_TO_TRIU = np.array([_TRIU_POS[p] for p in _BAND_PAIRS], dtype=np.int32)


@functools.cache
def _make_sc_gather():
    info = plsc.get_sparse_core_info()
    nw = info.num_cores * info.num_subcores
    total = B * N_SPARSE
    per_w = total // nw
    ch = 128
    n_ch = per_w // ch
    mesh = plsc.VectorSubcoreMesh(core_axis_name="c", subcore_axis_name="s")

    @functools.partial(
        pl.kernel,
        mesh=mesh,
        out_type=jax.ShapeDtypeStruct((total, K), jnp.float32),
        scratch_types=[
            pltpu.VMEM((per_w,), jnp.int32),
            pltpu.VMEM((2, ch, K), jnp.float32),
            pltpu.SemaphoreType.DMA,
            pltpu.SemaphoreType.DMA,
        ],
        compiler_params=pltpu.CompilerParams(use_tc_tiling_on_sc=False),
    )
    def gather_k(tab_hbm, idx_hbm, out_hbm, idx_v, rows_v, sem0, sem1):
        wid = lax.axis_index("s") * info.num_cores + lax.axis_index("c")
        base = wid * per_w
        pltpu.sync_copy(idx_hbm.at[pl.ds(base, per_w)], idx_v)
        sems = (sem0, sem1)

        def start(j, buf):
            pltpu.async_copy(
                tab_hbm.at[idx_v.at[pl.ds(j * ch, ch)]], rows_v.at[buf],
                sems[buf])

        def finish(j, buf):
            pltpu.make_async_copy(
                tab_hbm.at[idx_v.at[pl.ds(j * ch, ch)]], rows_v.at[buf],
                sems[buf]).wait()
            pltpu.sync_copy(rows_v.at[buf],
                            out_hbm.at[pl.ds(base + j * ch, ch)])

        start(0, 0)

        def body(p, _):
            j = 2 * p
            start(j + 1, 1)
            finish(j, 0)

            @pl.when(j + 2 < n_ch)
            def _():
                start(j + 2, 0)

            finish(j + 1, 1)
            return 0

        lax.fori_loop(0, n_ch // 2, body, 0)

    return gather_k


_M = 256


def _tc_body(xdT_ref, z_ref, w1_ref, b1_ref, w2_ref, b2_ref, out_ref):
    z = z_ref[...]
    zT = z.T
    xdT = xdT_ref[...]

    bands = []
    for d in range(1, N_SPARSE):
        w = (N_SPARSE - d) * K
        prod = zT[:w] * zT[d * K:]
        s = prod.reshape(N_SPARSE - d, K, _M).sum(axis=1)
        bands.append(s)
    ipT = jnp.concatenate(bands, axis=0)

    dT = jnp.concatenate([xdT, zT, ipT], axis=0)
    h = jnp.dot(w1_ref[...], dT, preferred_element_type=jnp.float32)
    h = jnp.maximum(h + b1_ref[...], 0.0)
    o = jnp.dot(w2_ref[...], h, preferred_element_type=jnp.float32)
    out_ref[...] = jax.nn.sigmoid(o + b2_ref[...])


@functools.cache
def _make_tc_mlp():
    grid = (B // _M,)
    return pl.pallas_call(
        _tc_body,
        grid=grid,
        in_specs=[
            pl.BlockSpec((N_DENSE, _M), lambda i: (0, i)),
            pl.BlockSpec((_M, SPARSE_DIMS), lambda i: (i, 0)),
            pl.BlockSpec((DNN_IN, DNN_IN), lambda i: (0, 0)),
            pl.BlockSpec((DNN_IN, 1), lambda i: (0, 0)),
            pl.BlockSpec((1, DNN_IN), lambda i: (0, 0)),
            pl.BlockSpec((1, 1), lambda i: (0, 0)),
        ],
        out_specs=pl.BlockSpec((1, _M), lambda i: (0, i)),
        out_shape=jax.ShapeDtypeStruct((1, B), jnp.float32),
    )


def kernel(x, tables, W1, b1, W2, b2):
    x_sparse = x[:, N_DENSE:]
    flat_idx = (x_sparse
                + (jnp.arange(N_SPARSE, dtype=jnp.int32) * VOCAB)[None, :])
    flat_idx = flat_idx.reshape(-1)
    tab_flat = tables.reshape(N_SPARSE * VOCAB, K)

    rows = _make_sc_gather()(tab_flat, flat_idx)
    z = rows.reshape(B, SPARSE_DIMS)

    xdT = x[:, :N_DENSE].astype(jnp.float32).T
    w1p = jnp.concatenate(
        [W1[:, : N_DENSE + SPARSE_DIMS],
         W1[:, N_DENSE + SPARSE_DIMS:][:, _BAND_TO_TRIU]], axis=1)

    out = _make_tc_mlp()(xdT, z, w1p, b1[:, None], W2, b2[:, None])
    return out.reshape(B, 1)


# speedup vs baseline: 1.0523x; 1.0139x over previous
import functools

import numpy as np
import jax
import jax.numpy as jnp
from jax import lax
from jax.experimental import pallas as pl
from jax.experimental.pallas import tpu as pltpu
from jax.experimental.pallas import tpu_sc as plsc

B = 16384
N_DENSE = 13
N_SPARSE = 26
K = 16
VOCAB = 100000
SPARSE_DIMS = N_SPARSE * K
N_PAIRS = N_SPARSE * (N_SPARSE - 1) // 2
DNN_IN = N_PAIRS + SPARSE_DIMS + N_DENSE

_row, _col = np.triu_indices(N_SPARSE, k=1)
_TRIU_POS = {(int(r), int(c)): t for t, (r, c) in enumerate(zip(_row, _col))}
_BAND_PAIRS = [(i, i + d) for d in range(1, N_SPARSE) for i in range(N_SPARSE - d)]
_BAND_TO_TRIU = np.array([_TRIU_POS[p] for p in _BAND_PAIRS], dtype=np.int32)


@functools.cache
def _make_sc_gather():
    info = plsc.get_sparse_core_info()
    nw = info.num_cores * info.num_subcores
    bw = B // nw
    mesh = plsc.VectorSubcoreMesh(core_axis_name="c", subcore_axis_name="s")

    @functools.partial(
        pl.kernel,
        mesh=mesh,
        out_type=jax.ShapeDtypeStruct((B, SPARSE_DIMS), jnp.float32),
        scratch_types=[
            pltpu.VMEM((2, bw), jnp.int32),
            pltpu.VMEM((2, bw, K), jnp.float32),
            pltpu.SemaphoreType.DMA,
            pltpu.SemaphoreType.DMA,
            pltpu.SemaphoreType.DMA,
            pltpu.SemaphoreType.DMA,
        ],
        compiler_params=pltpu.CompilerParams(use_tc_tiling_on_sc=False),
    )
    def gather_k(tab_hbm, idxT_hbm, out_hbm, idx_v, rows_v, gs0, gs1, os0, os1):
        wid = lax.axis_index("s") * info.num_cores + lax.axis_index("c")
        b0 = wid * bw
        gsem = (gs0, gs1)
        osem = (os0, os1)

        def start(f, buf):
            pltpu.sync_copy(idxT_hbm.at[f, pl.ds(b0, bw)], idx_v.at[buf])
            pltpu.async_copy(
                tab_hbm.at[f].at[idx_v.at[buf]], rows_v.at[buf], gsem[buf])

        def finish(f, buf):
            pltpu.make_async_copy(
                tab_hbm.at[f].at[idx_v.at[buf]], rows_v.at[buf],
                gsem[buf]).wait()
            pltpu.async_copy(
                rows_v.at[buf],
                out_hbm.at[pl.ds(b0, bw), pl.ds(f * K, K)], osem[buf])

        def wait_out(f, buf):
            pltpu.make_async_copy(
                rows_v.at[buf],
                out_hbm.at[pl.ds(b0, bw), pl.ds(f * K, K)], osem[buf]).wait()

        start(0, 0)
        start(1, 1)

        def body(p, _):
            f = 2 * p
            finish(f, 0)
            finish(f + 1, 1)

            @pl.when(f + 2 < N_SPARSE)
            def _():
                wait_out(f, 0)
                start(f + 2, 0)

            @pl.when(f + 3 < N_SPARSE)
            def _():
                wait_out(f + 1, 1)
                start(f + 3, 1)

            return 0

        lax.fori_loop(0, N_SPARSE // 2, body, 0)
        wait_out(N_SPARSE - 2, 0)
        wait_out(N_SPARSE - 1, 1)

    return gather_k


_M = 256


def _tc_body(xdT_ref, z_ref, w1_ref, b1_ref, w2_ref, b2_ref, out_ref):
    z = z_ref[...]
    zT = z.T
    xdT = xdT_ref[...]

    bands = []
    for d in range(1, N_SPARSE):
        w = (N_SPARSE - d) * K
        prod = zT[:w] * zT[d * K:]
        s = prod.reshape(N_SPARSE - d, K, _M).sum(axis=1)
        bands.append(s)
    ipT = jnp.concatenate(bands, axis=0)

    dT = jnp.concatenate([xdT, zT, ipT], axis=0)
    h = jnp.dot(w1_ref[...], dT, preferred_element_type=jnp.float32)
    h = jnp.maximum(h + b1_ref[...], 0.0)
    o = jnp.dot(w2_ref[...], h, preferred_element_type=jnp.float32)
    out_ref[...] = jax.nn.sigmoid(o + b2_ref[...])


@functools.cache
def _make_tc_mlp():
    grid = (B // _M,)
    return pl.pallas_call(
        _tc_body,
        grid=grid,
        in_specs=[
            pl.BlockSpec((N_DENSE, _M), lambda i: (0, i)),
            pl.BlockSpec((_M, SPARSE_DIMS), lambda i: (i, 0)),
            pl.BlockSpec((DNN_IN, DNN_IN), lambda i: (0, 0)),
            pl.BlockSpec((DNN_IN, 1), lambda i: (0, 0)),
            pl.BlockSpec((1, DNN_IN), lambda i: (0, 0)),
            pl.BlockSpec((1, 1), lambda i: (0, 0)),
        ],
        out_specs=pl.BlockSpec((1, _M), lambda i: (0, i)),
        out_shape=jax.ShapeDtypeStruct((1, B), jnp.float32),
    )


def kernel(x, tables, W1, b1, W2, b2):
    idxT = x[:, N_DENSE:].T
    z = _make_sc_gather()(tables, idxT)

    xdT = x[:, :N_DENSE].astype(jnp.float32).T
    w1p = jnp.concatenate(
        [W1[:, : N_DENSE + SPARSE_DIMS],
         W1[:, N_DENSE + SPARSE_DIMS:][:, _BAND_TO_TRIU]], axis=1)

    out = _make_tc_mlp()(xdT, z, w1p, b1[:, None], W2, b2[:, None])
    return out.reshape(B, 1)
